# baseline (speedup 1.0000x reference)
import functools

import jax
import jax.numpy as jnp
import numpy as np
from jax import lax
from jax.experimental import pallas as pl
from jax.experimental.pallas import tpu as pltpu
from jax.experimental.pallas import tpu_sc as plsc

_N = 10000
_E = 320000
_D = 128
_DH = _D // 2

_NC = 2
_NS = 16
_NW = _NC * _NS
_C = 128
_CH = 80
_EPW = _C * _CH
_EPAD = _NW * _EPW - _E
_RC = 80
_NRC = _N // _RC
_KMAX = (_NRC + _NS - 1) // _NS

_mesh = plsc.VectorSubcoreMesh(core_axis_name="c", subcore_axis_name="s")

_PI64 = np.concatenate([np.arange(0, 16), np.arange(32, 48),
                        np.arange(16, 32), np.arange(48, 64)])
_INV64 = np.argsort(_PI64)
_INV128 = np.concatenate([_INV64, _INV64 + _DH])


@functools.partial(
    pl.kernel,
    out_type=jax.ShapeDtypeStruct((2, _NC, _N, _DH), jnp.float32),
    mesh=_mesh,
    scratch_types=[
        pltpu.VMEM((_CH, _C), jnp.int32),
        pltpu.VMEM((_CH, _C), jnp.int32),
        pltpu.VMEM((_EPW,), jnp.float32),
        pltpu.VMEM((_C, _DH // 2), jnp.int32),
        pltpu.VMEM((_C, _DH // 2), jnp.int32),
        pltpu.VMEM((_C, _DH // 2), jnp.int32),
        pltpu.VMEM((_C, _DH // 2), jnp.int32),
        pltpu.VMEM((_C, _DH), jnp.float32),
        pltpu.VMEM((_C, _DH), jnp.float32),
        pltpu.VMEM((_RC, _DH), jnp.float32),
        pltpu.VMEM_SHARED((_N, _DH), jnp.float32),
        pltpu.SemaphoreType.DMA,
        pltpu.SemaphoreType.DMA,
        pltpu.SemaphoreType.DMA,
        pltpu.SemaphoreType.DMA,
        pltpu.SemaphoreType.DMA,
        pltpu.SemaphoreType.DMA,
    ],
    compiler_params=pltpu.CompilerParams(use_tc_tiling_on_sc=False,
                                         needs_layout_passes=False),
)
def _sc_aggregate(s_lo_hbm, s_hi_hbm, src_hbm, dst_hbm, wt_hbm, out_hbm,
                  src_v, dst_v, wt_v, rows0, rows1, rows2, rows3,
                  sc0, sc1, zero_v, acc_sh,
                  sem0, sem1, sem2, sem3, ssem0, ssem1):
    c = lax.axis_index("c")
    s = lax.axis_index("s")
    wid = s * _NC + c
    bufs = (rows0, rows1, rows2, rows3)
    sems = (sem0, sem1, sem2, sem3)
    sbufs = (sc0, sc1)
    ssems = (ssem0, ssem1)
    nb = len(bufs)
    ns = len(sbufs)

    pltpu.sync_copy(src_hbm.at[wid], src_v)
    pltpu.sync_copy(dst_hbm.at[wid], dst_v)
    pltpu.sync_copy(wt_hbm.at[wid], wt_v)

    def _zrow(i, carry):
        for d in range(_DH // 16):
            zero_v[i, pl.ds(d * 16, 16)] = jnp.zeros((16,), jnp.float32)
        return carry
    lax.fori_loop(0, _RC, _zrow, 0)

    def _scale(j, buf, sbuf):
        @plsc.parallel_loop(0, _C // 16, unroll=_C // 16)
        def _scale16(g):
            wvec = wt_v[pl.ds(j * _C + g * 16, 16)]
            for e in range(16):
                w16 = lax.broadcast(wvec[e], (16,))
                row = g * 16 + e
                for d in range(_DH // 32):
                    v = buf[row, pl.ds(d * 16, 16)]
                    a = plsc.bitcast(v << 16, jnp.float32)
                    b = plsc.bitcast(v & jnp.int32(-65536), jnp.float32)
                    sbuf[row, pl.ds(d * 32, 16)] = a * w16
                    sbuf[row, pl.ds(d * 32 + 16, 16)] = b * w16

    for ph, s_hbm in enumerate((s_lo_hbm, s_hi_hbm)):
        for k in range(_KMAX):
            idx = s + _NS * k
            @pl.when(idx < _NRC)
            def _():
                off = pl.multiple_of(idx * _RC, 8)
                pltpu.sync_copy(zero_v, acc_sh.at[pl.ds(off, _RC)])
        plsc.subcore_barrier()

        for b in range(nb):
            pltpu.async_copy(s_hbm.at[src_v.at[b]], bufs[b], sems[b])

        def _slot(j, b, sb):
            pltpu.make_async_copy(
                s_hbm.at[src_v.at[j]], bufs[b], sems[b]).wait()
            @pl.when(j >= ns)
            def _():
                pltpu.make_async_copy(
                    sbufs[sb], acc_sh.at[dst_v.at[j]], ssems[sb]).wait()
            _scale(j, bufs[b], sbufs[sb])
            pltpu.async_copy(
                sbufs[sb], acc_sh.at[dst_v.at[j]], ssems[sb], add=True)
            @pl.when(j + nb < _CH)
            def _():
                pltpu.async_copy(
                    s_hbm.at[src_v.at[j + nb]], bufs[b], sems[b])

        def _group(g, carry):
            for b in range(nb):
                j = g * nb + b
                _slot(j, b, b % ns)
            return carry
        lax.fori_loop(0, _CH // nb, _group, 0)
        for j in range((_CH // nb) * nb, _CH):
            _slot(j, j % nb, j % ns)
        for t in range(ns):
            jj = _CH - ns + t
            pltpu.make_async_copy(
                sbufs[jj % ns], acc_sh.at[dst_v.at[jj]],
                ssems[jj % ns]).wait()
        plsc.subcore_barrier()

        for k in range(_KMAX):
            idx = s + _NS * k
            @pl.when(idx < _NRC)
            def _():
                off = pl.multiple_of(idx * _RC, 8)
                pltpu.sync_copy(acc_sh.at[pl.ds(off, _RC)],
                                out_hbm.at[ph, c, pl.ds(off, _RC)])
        plsc.subcore_barrier()


def _pack_bf16_pairs(th):
    lo = th[:, :_DH // 2].astype(jnp.bfloat16).astype(jnp.float32)
    hi = th[:, _DH // 2:].astype(jnp.bfloat16).astype(jnp.float32)
    lo_b = jax.lax.bitcast_convert_type(lo, jnp.int32)
    hi_b = jax.lax.bitcast_convert_type(hi, jnp.int32)
    return jax.lax.shift_right_logical(lo_b, 16) | hi_b


def _mm_body(x_ref, w_ref, lo_ref, hi_ref):
    t = jnp.dot(x_ref[...], w_ref[...], preferred_element_type=jnp.float32)
    lo_ref[...] = _pack_bf16_pairs(t[:, :_DH])
    hi_ref[...] = _pack_bf16_pairs(t[:, _DH:])


def _tc_matmul_split(x, w):
    return pl.pallas_call(
        _mm_body,
        out_shape=(jax.ShapeDtypeStruct((x.shape[0], _DH // 2), jnp.int32),
                   jax.ShapeDtypeStruct((x.shape[0], _DH // 2), jnp.int32)),
    )(x, w)


def _cmb_body(p_ref, w_ref, lo_ref, hi_ref):
    h_lo = jnp.tanh(p_ref[0, 0] + p_ref[0, 1])
    h_hi = jnp.tanh(p_ref[1, 0] + p_ref[1, 1])
    h = jnp.concatenate([h_lo, h_hi], axis=1)
    t = jnp.dot(h, w_ref[...], preferred_element_type=jnp.float32)
    lo_ref[...] = _pack_bf16_pairs(t[:, :_DH])
    hi_ref[...] = _pack_bf16_pairs(t[:, _DH:])


def _tc_combine_matmul_split(p, w):
    return pl.pallas_call(
        _cmb_body,
        out_shape=(jax.ShapeDtypeStruct((_N, _DH // 2), jnp.int32),
                   jax.ShapeDtypeStruct((_N, _DH // 2), jnp.int32)),
    )(p, w)


def _fin_body(p_ref, o_ref):
    o_ref[:, :_DH] = jnp.tanh(p_ref[0, 0] + p_ref[0, 1])
    o_ref[:, _DH:] = jnp.tanh(p_ref[1, 0] + p_ref[1, 1])


def _tc_tanh_combine(p):
    return pl.pallas_call(
        _fin_body,
        out_shape=jax.ShapeDtypeStruct((_N, _D), jnp.float32),
    )(p)


def kernel(x, edge_index, edge_weight, W1, W2):
    zpad = jnp.zeros((_EPAD,), jnp.int32)
    src = jnp.concatenate([edge_index[0], zpad]).reshape(_NW, _CH, _C)
    dst = jnp.concatenate([edge_index[1], zpad]).reshape(_NW, _CH, _C)
    wt = jnp.concatenate(
        [edge_weight, jnp.zeros((_EPAD,), jnp.float32)]).reshape(_NW, _EPW)

    s1_lo, s1_hi = _tc_matmul_split(x, W1[:, _INV128])
    p1 = _sc_aggregate(s1_lo, s1_hi, src, dst, wt)
    s2_lo, s2_hi = _tc_combine_matmul_split(p1, W2[:, _INV128])
    p2 = _sc_aggregate(s2_lo, s2_hi, src, dst, wt)
    return _tc_tanh_combine(p2)


# speedup vs baseline: 1.7101x; 1.7101x over previous
import functools

import jax
import jax.numpy as jnp
import numpy as np
from jax import lax
from jax.experimental import pallas as pl
from jax.experimental.pallas import tpu as pltpu
from jax.experimental.pallas import tpu_sc as plsc

_N = 10000
_E = 320000
_D = 128
_DH = _D // 2

_NC = 2
_NS = 16
_NW = _NC * _NS
_C = 64
_CH = 157
_EPW = _C * _CH
_EPAD = _NW * _EPW - _E
_RC = 80
_NRC = _N // _RC
_KMAX = (_NRC + _NS - 1) // _NS

_mesh = plsc.VectorSubcoreMesh(core_axis_name="c", subcore_axis_name="s")

_PI64 = np.concatenate([np.arange(0, 16), np.arange(32, 48),
                        np.arange(16, 32), np.arange(48, 64)])
_INV64 = np.argsort(_PI64)
_INV128 = np.concatenate([_INV64, _INV64 + _DH])


@functools.partial(
    pl.kernel,
    out_type=jax.ShapeDtypeStruct((2, _NC, _N, _DH), jnp.float32),
    mesh=_mesh,
    scratch_types=[
        pltpu.VMEM((_CH, _C), jnp.int32),
        pltpu.VMEM((_CH, _C), jnp.int32),
        pltpu.VMEM((_EPW,), jnp.float32),
        pltpu.VMEM((_C, _DH // 2), jnp.int32),
        pltpu.VMEM((_C, _DH // 2), jnp.int32),
        pltpu.VMEM((_C, _DH // 2), jnp.int32),
        pltpu.VMEM((_C, _DH // 2), jnp.int32),
        pltpu.VMEM((_C, _DH), jnp.float32),
        pltpu.VMEM((_C, _DH), jnp.float32),
        pltpu.VMEM((_RC, _DH), jnp.float32),
        pltpu.VMEM_SHARED((_N, _DH), jnp.float32),
        pltpu.SemaphoreType.DMA,
        pltpu.SemaphoreType.DMA,
        pltpu.SemaphoreType.DMA,
        pltpu.SemaphoreType.DMA,
        pltpu.SemaphoreType.DMA,
        pltpu.SemaphoreType.DMA,
    ],
    compiler_params=pltpu.CompilerParams(use_tc_tiling_on_sc=False,
                                         needs_layout_passes=False),
)
def _sc_aggregate(s_lo_hbm, s_hi_hbm, src_hbm, dst_hbm, wt_hbm, out_hbm,
                  src_v, dst_v, wt_v, rows0, rows1, rows2, rows3,
                  sc0, sc1, zero_v, acc_sh,
                  sem0, sem1, sem2, sem3, ssem0, ssem1):
    c = lax.axis_index("c")
    s = lax.axis_index("s")
    wid = s * _NC + c
    bufs = (rows0, rows1, rows2, rows3)
    sems = (sem0, sem1, sem2, sem3)
    sbufs = (sc0, sc1)
    ssems = (ssem0, ssem1)
    nb = len(bufs)
    ns = len(sbufs)

    pltpu.sync_copy(src_hbm.at[wid], src_v)
    pltpu.sync_copy(dst_hbm.at[wid], dst_v)
    pltpu.sync_copy(wt_hbm.at[wid], wt_v)

    def _zrow(i, carry):
        for d in range(_DH // 16):
            zero_v[i, pl.ds(d * 16, 16)] = jnp.zeros((16,), jnp.float32)
        return carry
    lax.fori_loop(0, _RC, _zrow, 0)

    def _scale(j, buf, sbuf):
        @plsc.parallel_loop(0, _C // 16, unroll=_C // 16)
        def _scale16(g):
            wvec = wt_v[pl.ds(j * _C + g * 16, 16)]
            for e in range(16):
                w16 = lax.broadcast(wvec[e], (16,))
                row = g * 16 + e
                for d in range(_DH // 32):
                    v = buf[row, pl.ds(d * 16, 16)]
                    a = plsc.bitcast(v << 16, jnp.float32)
                    b = plsc.bitcast(v & jnp.int32(-65536), jnp.float32)
                    sbuf[row, pl.ds(d * 32, 16)] = a * w16
                    sbuf[row, pl.ds(d * 32 + 16, 16)] = b * w16

    for ph, s_hbm in enumerate((s_lo_hbm, s_hi_hbm)):
        for k in range(_KMAX):
            idx = s + _NS * k
            @pl.when(idx < _NRC)
            def _():
                off = pl.multiple_of(idx * _RC, 8)
                pltpu.sync_copy(zero_v, acc_sh.at[pl.ds(off, _RC)])
        plsc.subcore_barrier()

        for b in range(nb):
            pltpu.async_copy(s_hbm.at[src_v.at[b]], bufs[b], sems[b])

        def _slot(j, b, sb):
            pltpu.make_async_copy(
                s_hbm.at[src_v.at[j]], bufs[b], sems[b]).wait()
            @pl.when(j >= ns)
            def _():
                pltpu.make_async_copy(
                    sbufs[sb], acc_sh.at[dst_v.at[j]], ssems[sb]).wait()
            _scale(j, bufs[b], sbufs[sb])
            pltpu.async_copy(
                sbufs[sb], acc_sh.at[dst_v.at[j]], ssems[sb], add=True)
            @pl.when(j + nb < _CH)
            def _():
                pltpu.async_copy(
                    s_hbm.at[src_v.at[j + nb]], bufs[b], sems[b])

        def _group(g, carry):
            for b in range(nb):
                j = g * nb + b
                _slot(j, b, b % ns)
            return carry
        lax.fori_loop(0, _CH // nb, _group, 0)
        for j in range((_CH // nb) * nb, _CH):
            _slot(j, j % nb, j % ns)
        for t in range(ns):
            jj = _CH - ns + t
            pltpu.make_async_copy(
                sbufs[jj % ns], acc_sh.at[dst_v.at[jj]],
                ssems[jj % ns]).wait()
        plsc.subcore_barrier()

        for k in range(_KMAX):
            idx = s + _NS * k
            @pl.when(idx < _NRC)
            def _():
                off = pl.multiple_of(idx * _RC, 8)
                pltpu.sync_copy(acc_sh.at[pl.ds(off, _RC)],
                                out_hbm.at[ph, c, pl.ds(off, _RC)])
        plsc.subcore_barrier()


def _pack_bf16_pairs(th):
    lo = th[:, :_DH // 2].astype(jnp.bfloat16).astype(jnp.float32)
    hi = th[:, _DH // 2:].astype(jnp.bfloat16).astype(jnp.float32)
    lo_b = jax.lax.bitcast_convert_type(lo, jnp.int32)
    hi_b = jax.lax.bitcast_convert_type(hi, jnp.int32)
    return jax.lax.shift_right_logical(lo_b, 16) | hi_b


def _mm_body(x_ref, w_ref, lo_ref, hi_ref):
    t = jnp.dot(x_ref[...], w_ref[...], preferred_element_type=jnp.float32)
    lo_ref[...] = _pack_bf16_pairs(t[:, :_DH])
    hi_ref[...] = _pack_bf16_pairs(t[:, _DH:])


def _tc_matmul_split(x, w):
    return pl.pallas_call(
        _mm_body,
        out_shape=(jax.ShapeDtypeStruct((x.shape[0], _DH // 2), jnp.int32),
                   jax.ShapeDtypeStruct((x.shape[0], _DH // 2), jnp.int32)),
    )(x, w)


def _cmb_body(p_ref, w_ref, lo_ref, hi_ref):
    h_lo = jnp.tanh(p_ref[0, 0] + p_ref[0, 1])
    h_hi = jnp.tanh(p_ref[1, 0] + p_ref[1, 1])
    h = jnp.concatenate([h_lo, h_hi], axis=1)
    t = jnp.dot(h, w_ref[...], preferred_element_type=jnp.float32)
    lo_ref[...] = _pack_bf16_pairs(t[:, :_DH])
    hi_ref[...] = _pack_bf16_pairs(t[:, _DH:])


def _tc_combine_matmul_split(p, w):
    return pl.pallas_call(
        _cmb_body,
        out_shape=(jax.ShapeDtypeStruct((_N, _DH // 2), jnp.int32),
                   jax.ShapeDtypeStruct((_N, _DH // 2), jnp.int32)),
    )(p, w)


def _fin_body(p_ref, o_ref):
    o_ref[:, :_DH] = jnp.tanh(p_ref[0, 0] + p_ref[0, 1])
    o_ref[:, _DH:] = jnp.tanh(p_ref[1, 0] + p_ref[1, 1])


def _tc_tanh_combine(p):
    return pl.pallas_call(
        _fin_body,
        out_shape=jax.ShapeDtypeStruct((_N, _D), jnp.float32),
    )(p)


def kernel(x, edge_index, edge_weight, W1, W2):
    zpad = jnp.zeros((_EPAD,), jnp.int32)
    src = jnp.concatenate([edge_index[0], zpad]).reshape(_NW, _CH, _C)
    dst = jnp.concatenate([edge_index[1], zpad]).reshape(_NW, _CH, _C)
    wt = jnp.concatenate(
        [edge_weight, jnp.zeros((_EPAD,), jnp.float32)]).reshape(_NW, _EPW)

    s1_lo, s1_hi = _tc_matmul_split(x, W1[:, _INV128])
    p1 = _sc_aggregate(s1_lo, s1_hi, src, dst, wt)
    s2_lo, s2_hi = _tc_combine_matmul_split(p1, W2[:, _INV128])
    p2 = _sc_aggregate(s2_lo, s2_hi, src, dst, wt)
    return _tc_tanh_combine(p2)


# speedup vs baseline: 1.9606x; 1.1465x over previous
import functools

import jax
import jax.numpy as jnp
import numpy as np
from jax import lax
from jax.experimental import pallas as pl
from jax.experimental.pallas import tpu as pltpu
from jax.experimental.pallas import tpu_sc as plsc

_N = 10000
_E = 320000
_D = 128
_DH = _D // 2

_NC = 2
_NS = 16
_NW = _NC * _NS
_C = 80
_CH = 125
_EPW = _C * _CH
_EPAD = _NW * _EPW - _E
_RC = 80
_NRC = _N // _RC
_KMAX = (_NRC + _NS - 1) // _NS

_mesh = plsc.VectorSubcoreMesh(core_axis_name="c", subcore_axis_name="s")

_PI64 = np.concatenate([np.arange(0, 16), np.arange(32, 48),
                        np.arange(16, 32), np.arange(48, 64)])
_INV64 = np.argsort(_PI64)
_INV128 = np.concatenate([_INV64, _INV64 + _DH])


@functools.partial(
    pl.kernel,
    out_type=jax.ShapeDtypeStruct((2, _NC, _N, _DH), jnp.float32),
    mesh=_mesh,
    scratch_types=[
        pltpu.VMEM((_CH, _C), jnp.int32),
        pltpu.VMEM((_CH, _C), jnp.int32),
        pltpu.VMEM((_EPW,), jnp.float32),
        pltpu.VMEM((_C, _DH // 2), jnp.int32),
        pltpu.VMEM((_C, _DH // 2), jnp.int32),
        pltpu.VMEM((_C, _DH // 2), jnp.int32),
        pltpu.VMEM((_C, _DH // 2), jnp.int32),
        pltpu.VMEM((_C, _DH), jnp.float32),
        pltpu.VMEM((_C, _DH), jnp.float32),
        pltpu.VMEM((_RC, _DH), jnp.float32),
        pltpu.VMEM_SHARED((_N, _DH), jnp.float32),
        pltpu.SemaphoreType.DMA,
        pltpu.SemaphoreType.DMA,
        pltpu.SemaphoreType.DMA,
        pltpu.SemaphoreType.DMA,
        pltpu.SemaphoreType.DMA,
        pltpu.SemaphoreType.DMA,
    ],
    compiler_params=pltpu.CompilerParams(use_tc_tiling_on_sc=False,
                                         needs_layout_passes=False),
)
def _sc_aggregate(s_lo_hbm, s_hi_hbm, src_hbm, dst_hbm, wt_hbm, out_hbm,
                  src_v, dst_v, wt_v, rows0, rows1, rows2, rows3,
                  sc0, sc1, zero_v, acc_sh,
                  sem0, sem1, sem2, sem3, ssem0, ssem1):
    c = lax.axis_index("c")
    s = lax.axis_index("s")
    wid = s * _NC + c
    bufs = (rows0, rows1, rows2, rows3)
    sems = (sem0, sem1, sem2, sem3)
    sbufs = (sc0, sc1)
    ssems = (ssem0, ssem1)
    nb = len(bufs)
    ns = len(sbufs)

    pltpu.sync_copy(src_hbm.at[wid], src_v)
    pltpu.sync_copy(dst_hbm.at[wid], dst_v)
    pltpu.sync_copy(wt_hbm.at[wid], wt_v)

    def _zrow(i, carry):
        for d in range(_DH // 16):
            zero_v[i, pl.ds(d * 16, 16)] = jnp.zeros((16,), jnp.float32)
        return carry
    lax.fori_loop(0, _RC, _zrow, 0)

    def _scale(j, buf, sbuf):
        @plsc.parallel_loop(0, _C // 16, unroll=_C // 16)
        def _scale16(g):
            wvec = wt_v[pl.ds(j * _C + g * 16, 16)]
            for e in range(16):
                w16 = lax.broadcast(wvec[e], (16,))
                row = g * 16 + e
                for d in range(_DH // 32):
                    v = buf[row, pl.ds(d * 16, 16)]
                    a = plsc.bitcast(v << 16, jnp.float32)
                    b = plsc.bitcast(v & jnp.int32(-65536), jnp.float32)
                    sbuf[row, pl.ds(d * 32, 16)] = a * w16
                    sbuf[row, pl.ds(d * 32 + 16, 16)] = b * w16

    for ph, s_hbm in enumerate((s_lo_hbm, s_hi_hbm)):
        for k in range(_KMAX):
            idx = s + _NS * k
            @pl.when(idx < _NRC)
            def _():
                off = pl.multiple_of(idx * _RC, 8)
                pltpu.sync_copy(zero_v, acc_sh.at[pl.ds(off, _RC)])
        plsc.subcore_barrier()

        for b in range(nb):
            pltpu.async_copy(s_hbm.at[src_v.at[b]], bufs[b], sems[b])

        def _slot(j, b, sb):
            pltpu.make_async_copy(
                s_hbm.at[src_v.at[j]], bufs[b], sems[b]).wait()
            @pl.when(j >= ns)
            def _():
                pltpu.make_async_copy(
                    sbufs[sb], acc_sh.at[dst_v.at[j]], ssems[sb]).wait()
            _scale(j, bufs[b], sbufs[sb])
            pltpu.async_copy(
                sbufs[sb], acc_sh.at[dst_v.at[j]], ssems[sb], add=True)
            @pl.when(j + nb < _CH)
            def _():
                pltpu.async_copy(
                    s_hbm.at[src_v.at[j + nb]], bufs[b], sems[b])

        def _group(g, carry):
            for b in range(nb):
                j = g * nb + b
                _slot(j, b, b % ns)
            return carry
        lax.fori_loop(0, _CH // nb, _group, 0)
        for j in range((_CH // nb) * nb, _CH):
            _slot(j, j % nb, j % ns)
        for t in range(ns):
            jj = _CH - ns + t
            pltpu.make_async_copy(
                sbufs[jj % ns], acc_sh.at[dst_v.at[jj]],
                ssems[jj % ns]).wait()
        plsc.subcore_barrier()

        for k in range(_KMAX):
            idx = s + _NS * k
            @pl.when(idx < _NRC)
            def _():
                off = pl.multiple_of(idx * _RC, 8)
                pltpu.sync_copy(acc_sh.at[pl.ds(off, _RC)],
                                out_hbm.at[ph, c, pl.ds(off, _RC)])
        plsc.subcore_barrier()


def _pack_bf16_pairs(th):
    lo = th[:, :_DH // 2].astype(jnp.bfloat16).astype(jnp.float32)
    hi = th[:, _DH // 2:].astype(jnp.bfloat16).astype(jnp.float32)
    lo_b = jax.lax.bitcast_convert_type(lo, jnp.int32)
    hi_b = jax.lax.bitcast_convert_type(hi, jnp.int32)
    return jax.lax.shift_right_logical(lo_b, 16) | hi_b


def _mm_body(x_ref, w_ref, lo_ref, hi_ref):
    t = jnp.dot(x_ref[...], w_ref[...], preferred_element_type=jnp.float32)
    lo_ref[...] = _pack_bf16_pairs(t[:, :_DH])
    hi_ref[...] = _pack_bf16_pairs(t[:, _DH:])


def _tc_matmul_split(x, w):
    return pl.pallas_call(
        _mm_body,
        out_shape=(jax.ShapeDtypeStruct((x.shape[0], _DH // 2), jnp.int32),
                   jax.ShapeDtypeStruct((x.shape[0], _DH // 2), jnp.int32)),
    )(x, w)


def _cmb_body(p_ref, w_ref, lo_ref, hi_ref):
    h_lo = jnp.tanh(p_ref[0, 0] + p_ref[0, 1])
    h_hi = jnp.tanh(p_ref[1, 0] + p_ref[1, 1])
    h = jnp.concatenate([h_lo, h_hi], axis=1)
    t = jnp.dot(h, w_ref[...], preferred_element_type=jnp.float32)
    lo_ref[...] = _pack_bf16_pairs(t[:, :_DH])
    hi_ref[...] = _pack_bf16_pairs(t[:, _DH:])


def _tc_combine_matmul_split(p, w):
    return pl.pallas_call(
        _cmb_body,
        out_shape=(jax.ShapeDtypeStruct((_N, _DH // 2), jnp.int32),
                   jax.ShapeDtypeStruct((_N, _DH // 2), jnp.int32)),
    )(p, w)


def _fin_body(p_ref, o_ref):
    o_ref[:, :_DH] = jnp.tanh(p_ref[0, 0] + p_ref[0, 1])
    o_ref[:, _DH:] = jnp.tanh(p_ref[1, 0] + p_ref[1, 1])


def _tc_tanh_combine(p):
    return pl.pallas_call(
        _fin_body,
        out_shape=jax.ShapeDtypeStruct((_N, _D), jnp.float32),
    )(p)


def kernel(x, edge_index, edge_weight, W1, W2):
    zpad = jnp.zeros((_EPAD,), jnp.int32)
    src = jnp.concatenate([edge_index[0], zpad]).reshape(_NW, _CH, _C)
    dst = jnp.concatenate([edge_index[1], zpad]).reshape(_NW, _CH, _C)
    wt = jnp.concatenate(
        [edge_weight, jnp.zeros((_EPAD,), jnp.float32)]).reshape(_NW, _EPW)

    s1_lo, s1_hi = _tc_matmul_split(x, W1[:, _INV128])
    p1 = _sc_aggregate(s1_lo, s1_hi, src, dst, wt)
    s2_lo, s2_hi = _tc_combine_matmul_split(p1, W2[:, _INV128])
    p2 = _sc_aggregate(s2_lo, s2_hi, src, dst, wt)
    return _tc_tanh_combine(p2)
